# baseline (speedup 1.0000x reference)
import jax
import jax.numpy as jnp
from jax import lax
from jax.experimental import pallas as pl
from jax.experimental.pallas import tpu as pltpu
from jax.experimental.pallas import tpu_sc as plsc

N_U = 50000
N_I = 50000
N_E = 800000
D = 64

PAD_U = 50176
HALF = PAD_U // 2
STRIPE = HALF // 16
ACC_ROWS = HALF + 16
DUMMY = HALF
C = 128
SUP = 8
SUPE = C * SUP
RING = 3
EPAD = 2048
BM = 3136


def _mlp_norm_body(x_ref, w1_ref, b1_ref, w2_ref, b2_ref, o_ref):
    x = x_ref[...]
    h = jnp.dot(x, w1_ref[...], preferred_element_type=jnp.float32)
    h = jnp.maximum(h + b1_ref[...], 0.0)
    f = jnp.dot(h, w2_ref[...], preferred_element_type=jnp.float32)
    f = f + b2_ref[...]
    n = jnp.sqrt(jnp.sum(f * f, axis=-1, keepdims=True))
    o_ref[...] = f / jnp.maximum(n, 1e-12)


def _l2norm_body(x_ref, o_ref):
    x = x_ref[...]
    n = jnp.sqrt(jnp.sum(x * x, axis=-1, keepdims=True))
    o_ref[...] = x / jnp.maximum(n, 1e-12)


def _mlp_norm(x, W1, b1, W2, b2):
    n_rows = x.shape[0]
    k_in = W1.shape[0]
    hid = W1.shape[1]
    return pl.pallas_call(
        _mlp_norm_body,
        grid=(pl.cdiv(n_rows, BM),),
        in_specs=[
            pl.BlockSpec((BM, k_in), lambda i: (i, 0)),
            pl.BlockSpec((k_in, hid), lambda i: (0, 0)),
            pl.BlockSpec((1, hid), lambda i: (0, 0)),
            pl.BlockSpec((hid, D), lambda i: (0, 0)),
            pl.BlockSpec((1, D), lambda i: (0, 0)),
        ],
        out_specs=pl.BlockSpec((BM, D), lambda i: (i, 0)),
        out_shape=jax.ShapeDtypeStruct((n_rows, D), jnp.float32),
    )(x, W1, b1.reshape(1, -1), W2, b2.reshape(1, -1))


def _l2flat_body(x_ref, m_ref, o_ref):
    x = x_ref[...].reshape(-1, 128)
    n2 = jnp.dot(x * x, m_ref[...], preferred_element_type=jnp.float32)
    o = x / jnp.maximum(jnp.sqrt(n2), 1e-12)
    o_ref[...] = o.reshape(-1)


def _l2norm_flat(x_flat, blk_ones):
    bmf = 3584 * D
    return pl.pallas_call(
        _l2flat_body,
        grid=(pl.cdiv(x_flat.shape[0], bmf),),
        in_specs=[
            pl.BlockSpec((bmf,), lambda i: (i,)),
            pl.BlockSpec((128, 128), lambda i: (0, 0)),
        ],
        out_specs=pl.BlockSpec((bmf,), lambda i: (i,)),
        out_shape=jax.ShapeDtypeStruct((x_flat.shape[0],), jnp.float32),
    )(x_flat, blk_ones)


def _spmm_body(fn_hbm, col_hbm, row_hbm, bounds_hbm, out_hbm,
               bounds_v, colb_v, rowb_v, colr_v, locr_v, rows_v,
               acc_sh, gsem, ssem):
    c = lax.axis_index("c")
    s = lax.axis_index("s")
    w = c * 16 + s
    iota = lax.iota(jnp.int32, 16)

    def _zb(r, carry):
        for jj in range(D // 16):
            rows_v[0, r, pl.ds(jj * 16, 16)] = jnp.zeros((16,),
                                                         jnp.float32)
        return carry
    lax.fori_loop(0, C, _zb, 0)
    base_r = s * STRIPE
    zsrc = rows_v.at[0]
    for k in range(STRIPE // C):
        pltpu.sync_copy(zsrc, acc_sh.at[pl.ds(base_r + k * C, C)])
    rem = STRIPE % C
    if rem:
        pltpu.sync_copy(zsrc.at[pl.ds(0, rem)],
                        acc_sh.at[pl.ds(base_r + (STRIPE // C) * C, rem)])
    plsc.subcore_barrier()

    pltpu.sync_copy(bounds_hbm, bounds_v)
    e_start = bounds_v[pl.ds(w, 16)][0]
    e_end = bounds_v[pl.ds(32 + w, 16)][0]
    e0 = (e_start // 8) * 8
    nch = (e_end - e0 + (C - 1)) // C

    def _bulk(j):
        sb = e0 + (j // SUP) * SUPE
        pltpu.sync_copy(col_hbm.at[pl.ds(sb, SUPE)], colb_v)
        pltpu.sync_copy(row_hbm.at[pl.ds(sb, SUPE)], rowb_v)

    def _fix(j, slot):
        koff = lax.rem(j, SUP) * C
        base = e0 + j * C

        def _fi(i, cc):
            cv = colb_v[pl.ds(koff + i * 16, 16)]
            rv = rowb_v[pl.ds(koff + i * 16, 16)]
            eid = base + i * 16 + iota
            m = (eid >= e_start) & (eid < e_end)
            colr_v[slot, pl.ds(i * 16, 16)] = jnp.where(m, cv, 0)
            locr_v[slot, pl.ds(i * 16, 16)] = jnp.where(m, rv, DUMMY)
            return cc
        lax.fori_loop(0, C // 16, _fi, 0)

    def _wait_scatter(slot):
        pltpu.make_async_copy(rows_v.at[slot], acc_sh.at[locr_v.at[slot]],
                              ssem.at[slot]).wait()

    _bulk(jnp.int32(0))
    _fix(jnp.int32(0), 0)
    pltpu.async_copy(fn_hbm.at[colr_v.at[0]], rows_v.at[0], gsem.at[0])

    def _chunk(j, carry):
        sa = lax.rem(j, RING)
        sb = lax.rem(j + 1, RING)

        @pl.when(j >= 2)
        def _():
            _wait_scatter(sb)

        @pl.when(lax.rem(j + 1, SUP) == 0)
        def _():
            _bulk(j + 1)
        _fix(j + 1, sb)
        pltpu.async_copy(fn_hbm.at[colr_v.at[sb]], rows_v.at[sb],
                         gsem.at[sb])
        pltpu.make_async_copy(fn_hbm.at[colr_v.at[sa]], rows_v.at[sa],
                              gsem.at[sa]).wait()
        pltpu.async_copy(rows_v.at[sa], acc_sh.at[locr_v.at[sa]],
                         ssem.at[sa], add=True)
        return carry
    lax.fori_loop(0, nch, _chunk, 0)

    fin = lax.rem(nch, RING)
    pltpu.make_async_copy(fn_hbm.at[colr_v.at[fin]], rows_v.at[fin],
                          gsem.at[fin]).wait()

    @pl.when(nch > 1)
    def _():
        _wait_scatter(lax.rem(nch - 2, RING))

    @pl.when(nch > 0)
    def _():
        _wait_scatter(lax.rem(nch - 1, RING))
    plsc.subcore_barrier()

    sc_base = c * HALF
    pltpu.sync_copy(acc_sh.at[pl.ds(base_r, STRIPE)],
                    out_hbm.at[pl.ds(sc_base + base_r, STRIPE)])


_spmm = pl.kernel(
    _spmm_body,
    out_type=jax.ShapeDtypeStruct((PAD_U, D), jnp.float32),
    mesh=plsc.VectorSubcoreMesh(core_axis_name="c", subcore_axis_name="s"),
    compiler_params=pltpu.CompilerParams(use_tc_tiling_on_sc=False),
    scratch_types=[
        pltpu.VMEM((80,), jnp.int32),
        pltpu.VMEM((SUPE,), jnp.int32),
        pltpu.VMEM((SUPE,), jnp.int32),
        pltpu.VMEM((RING, C), jnp.int32),
        pltpu.VMEM((RING, C), jnp.int32),
        pltpu.VMEM((RING, C, D), jnp.float32),
        pltpu.VMEM_SHARED((ACC_ROWS, D), jnp.float32),
        pltpu.SemaphoreType.DMA((RING,)),
        pltpu.SemaphoreType.DMA((RING,)),
    ],
)


def kernel(teacher_input, W1, b1, W2, b2, edge_row, edge_col):
    fn = _mlp_norm(teacher_input, W1, b1, W2, b2)

    em = jnp.sum((edge_row < HALF).astype(jnp.int32)).astype(jnp.int32)
    t = jnp.arange(16, dtype=jnp.int32)
    sz0 = (em + 15) // 16
    sz1 = (N_E - em + 15) // 16
    s0 = jnp.minimum(t * sz0, em)
    e0 = jnp.minimum(s0 + sz0, em)
    s1 = jnp.minimum(em + t * sz1, N_E)
    e1 = jnp.minimum(s1 + sz1, N_E)
    bounds = jnp.concatenate(
        [s0, s1, e0, e1, jnp.zeros((16,), jnp.int32)]).astype(jnp.int32)

    rowloc = jnp.where(edge_row < HALF, edge_row, edge_row - HALF)
    colp = jnp.pad(edge_col, (0, EPAD))
    rowp = jnp.pad(rowloc, (0, EPAD))

    uv = _spmm(fn, colp, rowp, bounds)
    eye = jnp.arange(128, dtype=jnp.int32) // D
    blk_ones = (eye[:, None] == eye[None, :]).astype(jnp.float32)
    out1 = _l2norm_flat(uv.reshape(-1), blk_ones)
    return out1.reshape(PAD_U, D)[:N_U], fn


# speedup vs baseline: 1.0072x; 1.0072x over previous
import jax
import jax.numpy as jnp
from jax import lax
from jax.experimental import pallas as pl
from jax.experimental.pallas import tpu as pltpu
from jax.experimental.pallas import tpu_sc as plsc

N_U = 50000
N_I = 50000
N_E = 800000
D = 64

PAD_U = 50176
HALF = PAD_U // 2
STRIPE = HALF // 16
ACC_ROWS = HALF + 16
DUMMY = HALF
C = 128
SUP = 8
SUPE = C * SUP
RING = 3
EPAD = 2048
BM = 3136


def _mlp_norm_body(x_ref, w1_ref, b1_ref, w2_ref, b2_ref, o_ref):
    x = x_ref[...]
    h = jnp.dot(x, w1_ref[...], preferred_element_type=jnp.float32)
    h = jnp.maximum(h + b1_ref[...], 0.0)
    f = jnp.dot(h, w2_ref[...], preferred_element_type=jnp.float32)
    f = f + b2_ref[...]
    n = jnp.sqrt(jnp.sum(f * f, axis=-1, keepdims=True))
    o_ref[...] = f / jnp.maximum(n, 1e-12)


def _l2norm_body(x_ref, o_ref):
    x = x_ref[...]
    n = jnp.sqrt(jnp.sum(x * x, axis=-1, keepdims=True))
    o_ref[...] = x / jnp.maximum(n, 1e-12)


def _mlp_norm(x, W1, b1, W2, b2):
    n_rows = x.shape[0]
    k_in = W1.shape[0]
    hid = W1.shape[1]
    return pl.pallas_call(
        _mlp_norm_body,
        grid=(pl.cdiv(n_rows, BM),),
        in_specs=[
            pl.BlockSpec((BM, k_in), lambda i: (i, 0)),
            pl.BlockSpec((k_in, hid), lambda i: (0, 0)),
            pl.BlockSpec((1, hid), lambda i: (0, 0)),
            pl.BlockSpec((hid, D), lambda i: (0, 0)),
            pl.BlockSpec((1, D), lambda i: (0, 0)),
        ],
        out_specs=pl.BlockSpec((BM, D), lambda i: (i, 0)),
        out_shape=jax.ShapeDtypeStruct((n_rows, D), jnp.float32),
    )(x, W1, b1.reshape(1, -1), W2, b2.reshape(1, -1))


def _l2norm(x, n_out):
    bm = 3584
    return pl.pallas_call(
        _l2norm_body,
        grid=(pl.cdiv(x.shape[0], bm),),
        in_specs=[pl.BlockSpec((bm, D), lambda i: (i, 0))],
        out_specs=pl.BlockSpec((bm, D), lambda i: (i, 0)),
        out_shape=jax.ShapeDtypeStruct((n_out, D), jnp.float32),
    )(x)


def _spmm_body(fn_hbm, col_hbm, row_hbm, bounds_hbm, out_hbm,
               bounds_v, colb_v, rowb_v, colr_v, locr_v, rows_v,
               acc_sh, gsem, ssem):
    c = lax.axis_index("c")
    s = lax.axis_index("s")
    w = c * 16 + s
    iota = lax.iota(jnp.int32, 16)

    def _zb(r, carry):
        for jj in range(D // 16):
            rows_v[0, r, pl.ds(jj * 16, 16)] = jnp.zeros((16,),
                                                         jnp.float32)
        return carry
    lax.fori_loop(0, C, _zb, 0)
    base_r = s * STRIPE
    zsrc = rows_v.at[0]
    for k in range(STRIPE // C):
        pltpu.sync_copy(zsrc, acc_sh.at[pl.ds(base_r + k * C, C)])
    rem = STRIPE % C
    if rem:
        pltpu.sync_copy(zsrc.at[pl.ds(0, rem)],
                        acc_sh.at[pl.ds(base_r + (STRIPE // C) * C, rem)])
    plsc.subcore_barrier()

    pltpu.sync_copy(bounds_hbm, bounds_v)
    e_start = bounds_v[pl.ds(w, 16)][0]
    e_end = bounds_v[pl.ds(32 + w, 16)][0]
    e0 = (e_start // 8) * 8
    nch = (e_end - e0 + (C - 1)) // C

    def _bulk(j):
        sb = e0 + (j // SUP) * SUPE
        pltpu.sync_copy(col_hbm.at[pl.ds(sb, SUPE)], colb_v)
        pltpu.sync_copy(row_hbm.at[pl.ds(sb, SUPE)], rowb_v)

    def _fix(j, slot):
        koff = lax.rem(j, SUP) * C
        base = e0 + j * C

        def _fi(i, cc):
            cv = colb_v[pl.ds(koff + i * 16, 16)]
            rv = rowb_v[pl.ds(koff + i * 16, 16)]
            eid = base + i * 16 + iota
            m = (eid >= e_start) & (eid < e_end)
            colr_v[slot, pl.ds(i * 16, 16)] = jnp.where(m, cv, 0)
            locr_v[slot, pl.ds(i * 16, 16)] = jnp.where(m, rv, DUMMY)
            return cc
        lax.fori_loop(0, C // 16, _fi, 0)

    def _wait_scatter(slot):
        pltpu.make_async_copy(rows_v.at[slot], acc_sh.at[locr_v.at[slot]],
                              ssem.at[slot]).wait()

    _bulk(jnp.int32(0))
    _fix(jnp.int32(0), 0)
    pltpu.async_copy(fn_hbm.at[colr_v.at[0]], rows_v.at[0], gsem.at[0])

    def _chunk(j, carry):
        sa = lax.rem(j, RING)
        sb = lax.rem(j + 1, RING)

        @pl.when(j >= 2)
        def _():
            _wait_scatter(sb)

        @pl.when(lax.rem(j + 1, SUP) == 0)
        def _():
            _bulk(j + 1)
        _fix(j + 1, sb)
        pltpu.async_copy(fn_hbm.at[colr_v.at[sb]], rows_v.at[sb],
                         gsem.at[sb])
        pltpu.make_async_copy(fn_hbm.at[colr_v.at[sa]], rows_v.at[sa],
                              gsem.at[sa]).wait()
        pltpu.async_copy(rows_v.at[sa], acc_sh.at[locr_v.at[sa]],
                         ssem.at[sa], add=True)
        return carry
    lax.fori_loop(0, nch, _chunk, 0)

    fin = lax.rem(nch, RING)
    pltpu.make_async_copy(fn_hbm.at[colr_v.at[fin]], rows_v.at[fin],
                          gsem.at[fin]).wait()

    @pl.when(nch > 1)
    def _():
        _wait_scatter(lax.rem(nch - 2, RING))

    @pl.when(nch > 0)
    def _():
        _wait_scatter(lax.rem(nch - 1, RING))
    plsc.subcore_barrier()

    sc_base = c * HALF
    pltpu.sync_copy(acc_sh.at[pl.ds(base_r, STRIPE)],
                    out_hbm.at[pl.ds(sc_base + base_r, STRIPE)])


_spmm = pl.kernel(
    _spmm_body,
    out_type=jax.ShapeDtypeStruct((PAD_U, D), jnp.float32),
    mesh=plsc.VectorSubcoreMesh(core_axis_name="c", subcore_axis_name="s"),
    compiler_params=pltpu.CompilerParams(use_tc_tiling_on_sc=False),
    scratch_types=[
        pltpu.VMEM((80,), jnp.int32),
        pltpu.VMEM((SUPE,), jnp.int32),
        pltpu.VMEM((SUPE,), jnp.int32),
        pltpu.VMEM((RING, C), jnp.int32),
        pltpu.VMEM((RING, C), jnp.int32),
        pltpu.VMEM((RING, C, D), jnp.float32),
        pltpu.VMEM_SHARED((ACC_ROWS, D), jnp.float32),
        pltpu.SemaphoreType.DMA((RING,)),
        pltpu.SemaphoreType.DMA((RING,)),
    ],
)


def kernel(teacher_input, W1, b1, W2, b2, edge_row, edge_col):
    fn = _mlp_norm(teacher_input, W1, b1, W2, b2)

    em = jnp.sum((edge_row < HALF).astype(jnp.int32)).astype(jnp.int32)
    t = jnp.arange(16, dtype=jnp.int32)
    sz0 = (em + 15) // 16
    sz1 = (N_E - em + 15) // 16
    s0 = jnp.minimum(t * sz0, em)
    e0 = jnp.minimum(s0 + sz0, em)
    s1 = jnp.minimum(em + t * sz1, N_E)
    e1 = jnp.minimum(s1 + sz1, N_E)
    bounds = jnp.concatenate(
        [s0, s1, e0, e1, jnp.zeros((16,), jnp.int32)]).astype(jnp.int32)

    rowloc = jnp.where(edge_row < HALF, edge_row, edge_row - HALF)
    colp = jnp.pad(edge_col, (0, EPAD))
    rowp = jnp.pad(rowloc, (0, EPAD))

    uv = _spmm(fn, colp, rowp, bounds)
    out1 = _l2norm(uv, N_U)
    return out1, fn
